# baseline (speedup 1.0000x reference)
import functools

import jax
import jax.numpy as jnp
from jax import lax
from jax.experimental import pallas as pl
from jax.experimental.pallas import tpu as pltpu
from jax.experimental.pallas import tpu_sc as plsc

N, D = 50000, 128

BR = 2000
NBUF = 5
NB_TC = 20
NT = NB_TC // NBUF
N_TC = NB_TC * BR

NC, NS = 2, 16
NW = NC * NS
R = 160
GROUPS = R // 16
NBLOCKS = 2
PAIRS = NBLOCKS // 2
CHUNK = NBLOCKS * R

_GATHER_DNUMS = lax.GatherDimensionNumbers(
    offset_dims=(), collapsed_slice_dims=(0,), start_index_map=(0,))


def _perm(x, idx):
    return lax.gather(
        x, idx.reshape(16, 1), _GATHER_DNUMS, (1,),
        mode=lax.GatherScatterMode.PROMISE_IN_BOUNDS)


def _sc_body(gu_hbm, gi_hbm, out_hbm, ua, ia, ub, ib, obuf,
             sem_ua, sem_ia, sem_ub, sem_ib):
    wid = lax.axis_index("s") * NC + lax.axis_index("c")
    base_w = jnp.minimum(N_TC + wid * CHUNK, N - CHUNK)

    lanes = lax.iota(jnp.int32, 16)
    xor_idx = {h: lanes ^ h for h in (8, 4, 2, 1)}
    bitrev = (
        ((lanes & 1) << 3) | (((lanes >> 1) & 1) << 2)
        | (((lanes >> 2) & 1) << 1) | ((lanes >> 3) & 1))

    def combine(a, b, h):
        sel = (lanes & h) == 0
        return jnp.where(sel, a + _perm(a, xor_idx[h]), b + _perm(b, xor_idx[h]))

    def start(base, u_ref, i_ref, su, si):
        pltpu.async_copy(gu_hbm.at[pl.ds(base, R), :], u_ref, su)
        pltpu.async_copy(gi_hbm.at[pl.ds(base, R), :], i_ref, si)

    def wait(u_ref, i_ref, su, si):
        pltpu.make_async_copy(gu_hbm.at[pl.ds(0, R), :], u_ref, su).wait()
        pltpu.make_async_copy(gi_hbm.at[pl.ds(0, R), :], i_ref, si).wait()

    hs = (8, 4, 2, 1)

    def compute(u_ref, i_ref, off):
        def group(g, _):
            r0 = g * 16
            stack = []
            for rr in range(16):
                r = r0 + rr
                ps = [u_ref[r, pl.ds(k * 16, 16)] * i_ref[r, pl.ds(k * 16, 16)]
                      for k in range(8)]
                while len(ps) > 1:
                    ps = [ps[i] + ps[i + 1] for i in range(0, len(ps), 2)]
                node = (0, ps[0])
                while stack and stack[-1][0] == node[0]:
                    lvl, prev = stack.pop()
                    node = (lvl + 1, combine(prev, node[1], hs[lvl]))
                stack.append(node)
            obuf[pl.ds(off + r0, 16)] = _perm(stack[0][1], bitrev)
            return 0

        lax.fori_loop(0, GROUPS, group, 0)

    start(base_w, ua, ia, sem_ua, sem_ia)

    def pair(p, _):
        b0 = 2 * p
        start(base_w + (b0 + 1) * R, ub, ib, sem_ub, sem_ib)
        wait(ua, ia, sem_ua, sem_ia)
        compute(ua, ia, b0 * R)

        @pl.when(p < PAIRS - 1)
        def _():
            start(base_w + (b0 + 2) * R, ua, ia, sem_ua, sem_ia)

        wait(ub, ib, sem_ub, sem_ib)
        compute(ub, ib, (b0 + 1) * R)
        return 0

    lax.fori_loop(0, PAIRS, pair, 0)
    pltpu.sync_copy(obuf, out_hbm.at[pl.ds(base_w - N_TC, CHUNK)])


def _sc_call(gu, gi):
    f = functools.partial(
        pl.kernel,
        mesh=plsc.VectorSubcoreMesh(core_axis_name="c", subcore_axis_name="s"),
        out_type=jax.ShapeDtypeStruct((N - N_TC,), jnp.float32),
        scratch_types=[
            pltpu.VMEM((R, D), jnp.float32),
            pltpu.VMEM((R, D), jnp.float32),
            pltpu.VMEM((R, D), jnp.float32),
            pltpu.VMEM((R, D), jnp.float32),
            pltpu.VMEM((CHUNK,), jnp.float32),
            pltpu.SemaphoreType.DMA,
            pltpu.SemaphoreType.DMA,
            pltpu.SemaphoreType.DMA,
            pltpu.SemaphoreType.DMA,
        ],
    )(_sc_body)
    return f(gu, gi)


def _tc_body(gu_hbm, gi_hbm, o_ref, ubufs, ibufs, sem_u, sem_i):
    ones = jnp.ones((1, D), jnp.float32)

    def start(b, u):
        pltpu.async_copy(gu_hbm.at[pl.ds(b * BR, BR), :], ubufs.at[u], sem_u.at[u])
        pltpu.async_copy(gi_hbm.at[pl.ds(b * BR, BR), :], ibufs.at[u], sem_i.at[u])

    def wait(u):
        pltpu.make_async_copy(gu_hbm.at[pl.ds(0, BR), :], ubufs.at[u], sem_u.at[u]).wait()
        pltpu.make_async_copy(gi_hbm.at[pl.ds(0, BR), :], ibufs.at[u], sem_i.at[u]).wait()

    for u in range(NBUF):
        start(u, u)

    def outer(t, _):
        for u in range(NBUF):
            b = t * NBUF + u
            wait(u)
            prod = ubufs[u] * ibufs[u]
            o_ref[pl.ds(b, 1), :] = lax.dot_general(
                ones, prod, (((1,), (1,)), ((), ())),
                preferred_element_type=jnp.float32)

            @pl.when(t < NT - 1)
            def _():
                start(b + NBUF, u)
        return 0

    lax.fori_loop(0, NT, outer, 0)


def _tc_call(gu, gi):
    out = pl.pallas_call(
        _tc_body,
        in_specs=[
            pl.BlockSpec(memory_space=pltpu.MemorySpace.HBM),
            pl.BlockSpec(memory_space=pltpu.MemorySpace.HBM),
        ],
        out_shape=jax.ShapeDtypeStruct((NB_TC, BR), jnp.float32),
        scratch_shapes=[
            pltpu.VMEM((NBUF, BR, D), jnp.float32),
            pltpu.VMEM((NBUF, BR, D), jnp.float32),
            pltpu.SemaphoreType.DMA((NBUF,)),
            pltpu.SemaphoreType.DMA((NBUF,)),
        ],
    )(gu, gi)
    return out.reshape(N_TC)


@jax.jit
def kernel(gu, gi):
    out_sc = _sc_call(gu, gi)
    out_tc = _tc_call(gu, gi)
    return jnp.concatenate([out_tc, out_sc])


# speedup vs baseline: 1.1833x; 1.1833x over previous
import functools

import jax
import jax.numpy as jnp
from jax import lax
from jax.experimental import pallas as pl
from jax.experimental.pallas import tpu as pltpu
from jax.experimental.pallas import tpu_sc as plsc

N, D = 50000, 128

BR = 2048
NBUF = 11
NB_TC = 22
NT = NB_TC // NBUF
N_TC = NB_TC * BR

NC, NS = 2, 16
NW = NC * NS
R = 80
GROUPS = R // 16
NBLOCKS = 2
PAIRS = NBLOCKS // 2
CHUNK = NBLOCKS * R

_GATHER_DNUMS = lax.GatherDimensionNumbers(
    offset_dims=(), collapsed_slice_dims=(0,), start_index_map=(0,))


def _perm(x, idx):
    return lax.gather(
        x, idx.reshape(16, 1), _GATHER_DNUMS, (1,),
        mode=lax.GatherScatterMode.PROMISE_IN_BOUNDS)


def _sc_body(gu_hbm, gi_hbm, out_hbm, ua, ia, ub, ib, obuf,
             sem_ua, sem_ia, sem_ub, sem_ib):
    wid = lax.axis_index("s") * NC + lax.axis_index("c")
    base_w = jnp.minimum(N_TC + wid * CHUNK, N - CHUNK)

    lanes = lax.iota(jnp.int32, 16)
    xor_idx = {h: lanes ^ h for h in (8, 4, 2, 1)}
    bitrev = (
        ((lanes & 1) << 3) | (((lanes >> 1) & 1) << 2)
        | (((lanes >> 2) & 1) << 1) | ((lanes >> 3) & 1))

    def combine(a, b, h):
        sel = (lanes & h) == 0
        return jnp.where(sel, a + _perm(a, xor_idx[h]), b + _perm(b, xor_idx[h]))

    def start(base, u_ref, i_ref, su, si):
        pltpu.async_copy(gu_hbm.at[pl.ds(base, R), :], u_ref, su)
        pltpu.async_copy(gi_hbm.at[pl.ds(base, R), :], i_ref, si)

    def wait(u_ref, i_ref, su, si):
        pltpu.make_async_copy(gu_hbm.at[pl.ds(0, R), :], u_ref, su).wait()
        pltpu.make_async_copy(gi_hbm.at[pl.ds(0, R), :], i_ref, si).wait()

    hs = (8, 4, 2, 1)

    def compute(u_ref, i_ref, off):
        def group(g, _):
            r0 = g * 16
            stack = []
            for rr in range(16):
                r = r0 + rr
                ps = [u_ref[r, pl.ds(k * 16, 16)] * i_ref[r, pl.ds(k * 16, 16)]
                      for k in range(8)]
                while len(ps) > 1:
                    ps = [ps[i] + ps[i + 1] for i in range(0, len(ps), 2)]
                node = (0, ps[0])
                while stack and stack[-1][0] == node[0]:
                    lvl, prev = stack.pop()
                    node = (lvl + 1, combine(prev, node[1], hs[lvl]))
                stack.append(node)
            obuf[pl.ds(off + r0, 16)] = _perm(stack[0][1], bitrev)
            return 0

        lax.fori_loop(0, GROUPS, group, 0)

    start(base_w, ua, ia, sem_ua, sem_ia)

    def pair(p, _):
        b0 = 2 * p
        start(base_w + (b0 + 1) * R, ub, ib, sem_ub, sem_ib)
        wait(ua, ia, sem_ua, sem_ia)
        compute(ua, ia, b0 * R)

        @pl.when(p < PAIRS - 1)
        def _():
            start(base_w + (b0 + 2) * R, ua, ia, sem_ua, sem_ia)

        wait(ub, ib, sem_ub, sem_ib)
        compute(ub, ib, (b0 + 1) * R)
        return 0

    lax.fori_loop(0, PAIRS, pair, 0)
    pltpu.sync_copy(obuf, out_hbm.at[pl.ds(base_w - N_TC, CHUNK)])


def _sc_call(gu, gi):
    f = functools.partial(
        pl.kernel,
        mesh=plsc.VectorSubcoreMesh(core_axis_name="c", subcore_axis_name="s"),
        out_type=jax.ShapeDtypeStruct((N - N_TC,), jnp.float32),
        scratch_types=[
            pltpu.VMEM((R, D), jnp.float32),
            pltpu.VMEM((R, D), jnp.float32),
            pltpu.VMEM((R, D), jnp.float32),
            pltpu.VMEM((R, D), jnp.float32),
            pltpu.VMEM((CHUNK,), jnp.float32),
            pltpu.SemaphoreType.DMA,
            pltpu.SemaphoreType.DMA,
            pltpu.SemaphoreType.DMA,
            pltpu.SemaphoreType.DMA,
        ],
    )(_sc_body)
    return f(gu, gi)


def _tc_body(gu_hbm, gi_hbm, o_ref, ubufs, ibufs, sem_u, sem_i):
    ones = jnp.ones((1, D), jnp.float32)

    def start(b, u):
        pltpu.async_copy(gu_hbm.at[pl.ds(b * BR, BR), :], ubufs.at[u], sem_u.at[u])
        pltpu.async_copy(gi_hbm.at[pl.ds(b * BR, BR), :], ibufs.at[u], sem_i.at[u])

    def wait(u):
        pltpu.make_async_copy(gu_hbm.at[pl.ds(0, BR), :], ubufs.at[u], sem_u.at[u]).wait()
        pltpu.make_async_copy(gi_hbm.at[pl.ds(0, BR), :], ibufs.at[u], sem_i.at[u]).wait()

    for u in range(NBUF):
        start(u, u)

    def outer(t, _):
        for u in range(NBUF):
            b = t * NBUF + u
            wait(u)
            prod = ubufs[u] * ibufs[u]
            o_ref[pl.ds(b * BR, BR)] = lax.dot_general(
                ones, prod, (((1,), (1,)), ((), ())),
                preferred_element_type=jnp.float32).reshape(BR)

            @pl.when(t < NT - 1)
            def _():
                start(b + NBUF, u)
        return 0

    lax.fori_loop(0, NT, outer, 0)


def _tc_call(gu, gi):
    out = pl.pallas_call(
        _tc_body,
        in_specs=[
            pl.BlockSpec(memory_space=pltpu.MemorySpace.HBM),
            pl.BlockSpec(memory_space=pltpu.MemorySpace.HBM),
        ],
        out_shape=jax.ShapeDtypeStruct((N_TC,), jnp.float32),
        scratch_shapes=[
            pltpu.VMEM((NBUF, BR, D), jnp.float32),
            pltpu.VMEM((NBUF, BR, D), jnp.float32),
            pltpu.SemaphoreType.DMA((NBUF,)),
            pltpu.SemaphoreType.DMA((NBUF,)),
        ],
    )(gu, gi)
    return out


@jax.jit
def kernel(gu, gi):
    out_sc = _sc_call(gu, gi)
    out_tc = _tc_call(gu, gi)
    return jnp.concatenate([out_tc, out_sc])
